# baseline (speedup 1.0000x reference)
import functools

import jax
import jax.numpy as jnp
from jax import lax
from jax.experimental import pallas as pl
from jax.experimental.pallas import tpu as pltpu
from jax.experimental.pallas import tpu_sc as plsc

NUM_CORES = 2
NUM_SUBCORES = 16
NW = NUM_CORES * NUM_SUBCORES
LANES = 16
B = 16384
D = 64
TW = 2 * D
BPW = B // NW
CHUNK = 128
NCHUNK = BPW // CHUNK
PASS_CHUNKS = 2
ROWS_PER_PASS = PASS_CHUNKS * CHUNK
NPASS = NCHUNK // PASS_CHUNKS
BLK_PER_PASS = ROWS_PER_PASS // LANES
GRP_PER_PASS = ROWS_PER_PASS // LANES


def _mf_body(users_hbm, items_hbm, tab_u_hbm, tab_i_hbm, drain_hbm, out_hbm,
             idx_u, idx_i, rows_u, rows_i, out_v, sem):
    wid = lax.axis_index("c") * NUM_SUBCORES + lax.axis_index("s")
    base = wid * BPW

    pltpu.sync_copy(users_hbm.at[wid], idx_u)
    pltpu.sync_copy(items_hbm.at[wid], idx_i)

    iota = lax.iota(jnp.int32, LANES)
    kpg = CHUNK // LANES
    for p in range(NPASS):
        def row_dmas(v, carry, p=p):
            c = p * PASS_CHUNKS + v // kpg
            s = pl.ds((v % kpg) * LANES, LANES)
            iv_u = idx_u[c, s]
            iv_i = idx_i[c, s]
            for l in range(LANES):
                r = v * (LANES // 2) + (l >> 1)
                dst = pl.ds((l & 1) * D, D)
                pltpu.async_copy(tab_u_hbm.at[iv_u[l]], rows_u.at[r, dst], sem)
                pltpu.async_copy(tab_i_hbm.at[iv_i[l]], rows_i.at[r, dst], sem)
            return carry

        lax.fori_loop(0, GRP_PER_PASS, row_dmas, 0)
        pltpu.make_async_copy(drain_hbm, rows_u, sem).wait()
        pltpu.make_async_copy(drain_hbm, rows_i, sem).wait()

        cb = lax.shift_left(iota & 1, 6)
        half_ids = lax.shift_right_logical(iota, 1)

        def dot_blk(blk, carry, p=p):
            g = p * ROWS_PER_PASS + blk * LANES
            row_ids = blk * (LANES // 2) + half_ids
            acc = jnp.zeros((LANES,), jnp.float32)
            for d in range(D):
                col = cb + d
                u = plsc.load_gather(rows_u, [row_ids, col])
                v = plsc.load_gather(rows_i, [row_ids, col])
                acc = acc + u * v
            out_v[pl.ds(g, LANES)] = 1.0 / (1.0 + jnp.exp(-acc))
            return carry

        lax.fori_loop(0, BLK_PER_PASS, dot_blk, 0)

    pltpu.sync_copy(out_v, out_hbm.at[pl.ds(base, BPW)])


@jax.jit
def _mf_call(users_r, items_r, tab_u, tab_i, drain_src):
    mesh = plsc.VectorSubcoreMesh(core_axis_name="c", subcore_axis_name="s")
    run = functools.partial(
        pl.kernel,
        mesh=mesh,
        out_type=jax.ShapeDtypeStruct((B,), jnp.float32),
        scratch_types=[
            pltpu.VMEM((NCHUNK, CHUNK), jnp.int32),
            pltpu.VMEM((NCHUNK, CHUNK), jnp.int32),
            pltpu.VMEM((ROWS_PER_PASS // 2, TW), jnp.float32),
            pltpu.VMEM((ROWS_PER_PASS // 2, TW), jnp.float32),
            pltpu.VMEM((BPW,), jnp.float32),
            pltpu.SemaphoreType.DMA,
        ],
        compiler_params=pltpu.CompilerParams(
            needs_layout_passes=False, use_tc_tiling_on_sc=False),
    )(_mf_body)
    return run(users_r, items_r, tab_u, tab_i, drain_src)


def kernel(users, items, embedding_user, embedding_item):
    users_r = users.reshape(NW, NCHUNK, CHUNK)
    items_r = items.reshape(NW, NCHUNK, CHUNK)
    drain_src = lax.bitcast_convert_type(users, jnp.float32).reshape(
        ROWS_PER_PASS // 2, TW)
    return _mf_call(users_r, items_r, embedding_user, embedding_item,
                    drain_src)


# speedup vs baseline: 1.5650x; 1.5650x over previous
import functools

import jax
import jax.numpy as jnp
from jax import lax
from jax.experimental import pallas as pl
from jax.experimental.pallas import tpu as pltpu
from jax.experimental.pallas import tpu_sc as plsc

NUM_CORES = 2
NUM_SUBCORES = 16
NW = NUM_CORES * NUM_SUBCORES
LANES = 16
B = 16384
D = 64
TW = 2 * D
BPW = B // NW
CHUNK = 128
NCHUNK = BPW // CHUNK
PASS_CHUNKS = 2
ROWS_PER_PASS = PASS_CHUNKS * CHUNK
NPASS = NCHUNK // PASS_CHUNKS
BLK_PER_PASS = ROWS_PER_PASS // LANES
GRP_PER_PASS = ROWS_PER_PASS // LANES


def _mf_body(users_hbm, items_hbm, tab_u_hbm, tab_i_hbm, drain_hbm, out_hbm,
             idx_u, idx_i, rows_u, rows_i, out_v, sem):
    wid = lax.axis_index("c") * NUM_SUBCORES + lax.axis_index("s")
    base = wid * BPW

    pltpu.sync_copy(users_hbm.at[wid], idx_u)
    pltpu.sync_copy(items_hbm.at[wid], idx_i)

    iota = lax.iota(jnp.int32, LANES)
    kpg = CHUNK // LANES
    for p in range(NPASS):
        def row_dmas(v, carry, p=p):
            c = p * PASS_CHUNKS + v // kpg
            s = pl.ds((v % kpg) * LANES, LANES)
            iv_u = idx_u[c, s]
            iv_i = idx_i[c, s]
            for l in range(LANES):
                r = v * (LANES // 2) + (l >> 1)
                dst = pl.ds((l & 1) * D, D)
                pltpu.async_copy(tab_u_hbm.at[iv_u[l]], rows_u.at[r, dst], sem)
                pltpu.async_copy(tab_i_hbm.at[iv_i[l]], rows_i.at[r, dst], sem)
            return carry

        lax.fori_loop(0, GRP_PER_PASS, row_dmas, 0)
        pltpu.make_async_copy(drain_hbm, rows_u, sem).wait()
        pltpu.make_async_copy(drain_hbm, rows_i, sem).wait()

        cb = lax.shift_left(iota & 1, 6)
        half_ids = lax.shift_right_logical(iota, 1)

        def dot_blk(blk, carry, p=p):
            g = p * ROWS_PER_PASS + blk * LANES
            row_ids = blk * (LANES // 2) + half_ids
            acc = jnp.zeros((LANES,), jnp.float32)
            for d in range(D):
                col = cb + d
                u = plsc.load_gather(rows_u, [row_ids, col])
                v = plsc.load_gather(rows_i, [row_ids, col])
                acc = acc + u * v
            out_v[pl.ds(g, LANES)] = 1.0 / (1.0 + jnp.exp(-acc))
            return carry

        lax.fori_loop(0, BLK_PER_PASS, dot_blk, 0)

    pltpu.sync_copy(out_v, out_hbm.at[pl.ds(base, BPW)])


@jax.jit
def _mf_call(users_r, items_r, tab_u, tab_i, drain_src):
    mesh = plsc.VectorSubcoreMesh(core_axis_name="c", subcore_axis_name="s")
    run = functools.partial(
        pl.kernel,
        mesh=mesh,
        out_type=jax.ShapeDtypeStruct((B,), jnp.float32),
        scratch_types=[
            pltpu.VMEM((NCHUNK, CHUNK), jnp.int32),
            pltpu.VMEM((NCHUNK, CHUNK), jnp.int32),
            pltpu.VMEM((ROWS_PER_PASS // 2, TW), jnp.float32),
            pltpu.VMEM((ROWS_PER_PASS // 2, TW), jnp.float32),
            pltpu.VMEM((BPW,), jnp.float32),
            pltpu.SemaphoreType.DMA,
        ],
        compiler_params=pltpu.CompilerParams(needs_layout_passes=False),
    )(_mf_body)
    return run(users_r, items_r, tab_u, tab_i, drain_src)


def kernel(users, items, embedding_user, embedding_item):
    users_r = users.reshape(NW, NCHUNK, CHUNK)
    items_r = items.reshape(NW, NCHUNK, CHUNK)
    drain_src = lax.bitcast_convert_type(users, jnp.float32).reshape(
        ROWS_PER_PASS // 2, TW)
    return _mf_call(users_r, items_r, embedding_user, embedding_item,
                    drain_src)
